# Initial kernel scaffold; baseline (speedup 1.0000x reference)
import jax
import jax.numpy as jnp
from jax.experimental import pallas as pl


def kernel(x, layers):
    raise NotImplementedError("write your pallas kernel here")


# speedup vs baseline: 6.8683x; 6.8683x over previous
import jax
import jax.numpy as jnp
from jax.experimental import pallas as pl
from jax.experimental.pallas import tpu as pltpu

N = 40
G = 8
M = G * N
CIN_P = 8
NEG = -1e30

_PREC = jax.lax.Precision.HIGHEST


def _conv(h, W, asv_row, adv_row, bias_row, neg_mask):
    hp = jnp.dot(h, W, preferred_element_type=jnp.float32, precision=_PREC)
    a_s = jax.lax.dot_general(hp, asv_row, (((1,), (1,)), ((), ())),
                              preferred_element_type=jnp.float32,
                              precision=_PREC)
    a_d = jax.lax.dot_general(adv_row, hp, (((1,), (1,)), ((), ())),
                              preferred_element_type=jnp.float32,
                              precision=_PREC)
    logits = a_s + a_d
    logits = jnp.where(logits > 0, logits, 0.2 * logits)
    logits = logits + neg_mask
    m = jnp.max(logits, axis=0, keepdims=True)
    ex = jnp.exp(logits - m)
    den = jnp.sum(ex, axis=0, keepdims=True)
    att = ex / den
    out = jax.lax.dot_general(att, hp, (((0,), (0,)), ((), ())),
                              preferred_element_type=jnp.float32,
                              precision=_PREC)
    return out + bias_row


def _gat_body(feats_ref, w0_ref, ws_ref, asrc_ref, adst_ref, bias_ref, out_ref):
    row_g = jax.lax.broadcasted_iota(jnp.int32, (M, M), 0) // N
    col_g = jax.lax.broadcasted_iota(jnp.int32, (M, M), 1) // N
    neg_mask = jnp.where(row_g == col_g, 0.0, NEG).astype(jnp.float32)

    def layer(l, h, W):
        asv = asrc_ref[pl.ds(l, 1), :]
        adv = adst_ref[pl.ds(l, 1), :]
        bv = bias_ref[pl.ds(l, 1), :]
        return _conv(h, W, asv, adv, bv, neg_mask)

    feats = feats_ref[...]
    h = jax.nn.sigmoid(layer(0, feats, w0_ref[...]))
    for l in range(1, 4):
        h = h + jax.nn.sigmoid(layer(l, h, ws_ref[l - 1]))
    h = layer(4, h, ws_ref[3])

    srow = jax.lax.broadcasted_iota(jnp.int32, (G, M), 0)
    scol = jax.lax.broadcasted_iota(jnp.int32, (G, M), 1) // N
    sel = (srow == scol).astype(jnp.float32)
    out_ref[...] = jnp.dot(sel, h, preferred_element_type=jnp.float32,
                           precision=_PREC)


def kernel(x, layers):
    B = x.shape[0]
    feats = x[:, 120:].reshape(B * N, 3)
    feats = jnp.pad(feats, ((0, 0), (0, CIN_P - 3)))
    w0 = jnp.pad(layers[0][0], ((0, CIN_P - 3), (0, 0)))
    ws = jnp.stack([layers[i][0] for i in range(1, 5)])
    asrc = jnp.stack([layers[i][1] for i in range(5)])
    adst = jnp.stack([layers[i][2] for i in range(5)])
    bias = jnp.stack([layers[i][3] for i in range(5)])

    grid = (B // G,)
    out = pl.pallas_call(
        _gat_body,
        grid=grid,
        in_specs=[
            pl.BlockSpec((M, CIN_P), lambda i: (i, 0)),
            pl.BlockSpec((CIN_P, 64), lambda i: (0, 0)),
            pl.BlockSpec((4, 64, 64), lambda i: (0, 0, 0)),
            pl.BlockSpec((5, 64), lambda i: (0, 0)),
            pl.BlockSpec((5, 64), lambda i: (0, 0)),
            pl.BlockSpec((5, 64), lambda i: (0, 0)),
        ],
        out_specs=pl.BlockSpec((G, 64), lambda i: (i, 0)),
        out_shape=jax.ShapeDtypeStruct((B, 64), jnp.float32),
        compiler_params=pltpu.CompilerParams(
            dimension_semantics=("arbitrary",),
        ),
    )(feats, w0, ws, asrc, adst, bias)
    return out
